# Initial kernel scaffold; baseline (speedup 1.0000x reference)
import jax
import jax.numpy as jnp
from jax.experimental import pallas as pl


def kernel(r, species_neighbor, w_comb, mlp_w1, mlp_w2, mlp_w3, mlp_w4):
    raise NotImplementedError("write your pallas kernel here")


# speedup vs baseline: 1.2550x; 1.2550x over previous
import functools

import jax
import jax.numpy as jnp
from jax.experimental import pallas as pl

_R_CUT = 5.0
_L = 4
_A = 4
_NSP = 8
_NTOT = 128


def _silu(x):
    return x * jax.nn.sigmoid(x)


def _fwd(r_ref, s_ref, wc_ref, w1_ref, w2_ref, w3_ref, w4_ref, out_ref):
    eb = r_ref.shape[0]
    x = r_ref[:, :]
    xc = jnp.clip(x, 0.0, _R_CUT)
    cutoff = 0.5 * (jnp.cos((jnp.pi / _R_CUT) * xc) + 1.0)
    ki = jax.lax.broadcasted_iota(jnp.int32, (eb, _NTOT), 1) + 1
    k = ki.astype(jnp.float32)
    rf = jnp.sin((jnp.pi / _R_CUT) * (k * xc)) * cutoff

    s = s_ref[:, :]
    sp = jax.lax.broadcasted_iota(jnp.int32, (eb, _NSP), 1)
    onehot = (s == sp).astype(jnp.float32)
    psw = jnp.dot(onehot, wc_ref[:, :].T,
                  preferred_element_type=jnp.float32)

    for a in range(_A):
        g = psw[:, a][:, None]
        z = jnp.dot(rf, w1_ref[a], preferred_element_type=jnp.float32)
        h = _silu(z * g)
        h = _silu(jnp.dot(h, w2_ref[a], preferred_element_type=jnp.float32))
        h = _silu(jnp.dot(h, w3_ref[a], preferred_element_type=jnp.float32))
        o = jnp.dot(h, w4_ref[a], preferred_element_type=jnp.float32)
        out_ref[:, a, :] = o


def _block_diag_t(w):
    m = jnp.zeros((_A, _NTOT, _NTOT), jnp.float32)
    for l in range(_L):
        blk = jnp.transpose(w[l], (0, 2, 1))
        m = m.at[:, 32 * l:32 * (l + 1), 32 * l:32 * (l + 1)].set(blk)
    return m


@functools.partial(jax.jit, static_argnames=())
def kernel(r, species_neighbor, w_comb, mlp_w1, mlp_w2, mlp_w3, mlp_w4):
    e = r.shape[0]
    eb = 1280
    grid = pl.cdiv(e, eb)
    w1 = _block_diag_t(mlp_w1)
    w2 = _block_diag_t(mlp_w2)
    w3 = _block_diag_t(mlp_w3)
    w4 = _block_diag_t(mlp_w4)
    r2 = r.astype(jnp.float32).reshape(e, 1)
    s2 = species_neighbor.astype(jnp.int32).reshape(e, 1)
    full = lambda i: (0, 0, 0)
    return pl.pallas_call(
        _fwd,
        grid=(grid,),
        in_specs=[
            pl.BlockSpec((eb, 1), lambda i: (i, 0)),
            pl.BlockSpec((eb, 1), lambda i: (i, 0)),
            pl.BlockSpec((_A, _NSP), lambda i: (0, 0)),
            pl.BlockSpec((_A, _NTOT, _NTOT), full),
            pl.BlockSpec((_A, _NTOT, _NTOT), full),
            pl.BlockSpec((_A, _NTOT, _NTOT), full),
            pl.BlockSpec((_A, _NTOT, _NTOT), full),
        ],
        out_specs=pl.BlockSpec((eb, _A, _NTOT), lambda i: (i, 0, 0)),
        out_shape=jax.ShapeDtypeStruct((e, _A, _NTOT), jnp.float32),
    )(r2, s2, w_comb, w1, w2, w3, w4)


# speedup vs baseline: 2.0419x; 1.6271x over previous
import functools

import jax
import jax.numpy as jnp
from jax.experimental import pallas as pl

_R_CUT = 5.0
_L = 4
_A = 4
_NSP = 8
_NTOT = 128


def _silu(x):
    return x / (1.0 + jnp.exp(-x))


def _sinpi(t):
    t2 = t * t
    p = jnp.float32(-0.007370430945714351)
    p = p * t2 + jnp.float32(0.08214588661112823)
    p = p * t2 + jnp.float32(-0.5992645293207921)
    p = p * t2 + jnp.float32(2.550164039877345)
    p = p * t2 + jnp.float32(-5.16771278004997)
    p = p * t2 + jnp.float32(3.141592653589793)
    return p * t


def _fwd(r_ref, s_ref, wc_ref, w1_ref, w2_ref, w3_ref, w4_ref, out_ref):
    eb = r_ref.shape[0]
    x = r_ref[:, :]
    u = jnp.clip(x, 0.0, _R_CUT) * jnp.float32(1.0 / _R_CUT)
    cutoff = 0.5 * (_sinpi(0.5 - u) + 1.0)
    ki = jax.lax.broadcasted_iota(jnp.int32, (eb, _NTOT), 1) + 1
    k = ki.astype(jnp.float32)
    ku = k * u
    n = jnp.floor(ku + 0.5)
    f = ku - n
    half = n * 0.5
    sgn = 1.0 - 4.0 * (half - jnp.floor(half))
    rf = _sinpi(f) * (sgn * cutoff)

    s = s_ref[:, :]
    sp = jax.lax.broadcasted_iota(jnp.int32, (eb, _NSP), 1)
    onehot = (s == sp).astype(jnp.float32)
    psw = jnp.dot(onehot, wc_ref[:, :].T,
                  preferred_element_type=jnp.float32)

    for a in range(_A):
        g = psw[:, a][:, None]
        z = jnp.dot(rf, w1_ref[a], preferred_element_type=jnp.float32)
        h = _silu(z * g)
        h = _silu(jnp.dot(h, w2_ref[a], preferred_element_type=jnp.float32))
        h = _silu(jnp.dot(h, w3_ref[a], preferred_element_type=jnp.float32))
        o = jnp.dot(h, w4_ref[a], preferred_element_type=jnp.float32)
        out_ref[:, a, :] = o


def _block_diag_t(w):
    m = jnp.zeros((_A, _NTOT, _NTOT), jnp.float32)
    for l in range(_L):
        blk = jnp.transpose(w[l], (0, 2, 1))
        m = m.at[:, 32 * l:32 * (l + 1), 32 * l:32 * (l + 1)].set(blk)
    return m


@functools.partial(jax.jit, static_argnames=())
def kernel(r, species_neighbor, w_comb, mlp_w1, mlp_w2, mlp_w3, mlp_w4):
    e = r.shape[0]
    eb = 1280
    grid = pl.cdiv(e, eb)
    w1 = _block_diag_t(mlp_w1)
    w2 = _block_diag_t(mlp_w2)
    w3 = _block_diag_t(mlp_w3)
    w4 = _block_diag_t(mlp_w4)
    r2 = r.astype(jnp.float32).reshape(e, 1)
    s2 = species_neighbor.astype(jnp.int32).reshape(e, 1)
    full = lambda i: (0, 0, 0)
    return pl.pallas_call(
        _fwd,
        grid=(grid,),
        in_specs=[
            pl.BlockSpec((eb, 1), lambda i: (i, 0)),
            pl.BlockSpec((eb, 1), lambda i: (i, 0)),
            pl.BlockSpec((_A, _NSP), lambda i: (0, 0)),
            pl.BlockSpec((_A, _NTOT, _NTOT), full),
            pl.BlockSpec((_A, _NTOT, _NTOT), full),
            pl.BlockSpec((_A, _NTOT, _NTOT), full),
            pl.BlockSpec((_A, _NTOT, _NTOT), full),
        ],
        out_specs=pl.BlockSpec((eb, _A, _NTOT), lambda i: (i, 0, 0)),
        out_shape=jax.ShapeDtypeStruct((e, _A, _NTOT), jnp.float32),
    )(r2, s2, w_comb, w1, w2, w3, w4)


# speedup vs baseline: 2.1255x; 1.0409x over previous
import functools

import jax
import jax.numpy as jnp
from jax.experimental import pallas as pl

_R_CUT = 5.0
_L = 4
_A = 4
_NSP = 8
_NTOT = 128


def _silu(x):
    y = 0.5 * x
    return y * jnp.tanh(y) + y


def _sinpi(t):
    t2 = t * t
    p = jnp.float32(-0.007370430945714351)
    p = p * t2 + jnp.float32(0.08214588661112823)
    p = p * t2 + jnp.float32(-0.5992645293207921)
    p = p * t2 + jnp.float32(2.550164039877345)
    p = p * t2 + jnp.float32(-5.16771278004997)
    p = p * t2 + jnp.float32(3.141592653589793)
    return p * t


def _fwd(r_ref, s_ref, wc_ref, w1_ref, w2_ref, w3_ref, w4_ref, out_ref):
    eb = r_ref.shape[0]
    x = r_ref[:, :]
    u = jnp.clip(x, 0.0, _R_CUT) * jnp.float32(1.0 / _R_CUT)
    cutoff = 0.5 * (_sinpi(0.5 - u) + 1.0)
    ki = jax.lax.broadcasted_iota(jnp.int32, (eb, _NTOT), 1) + 1
    k = ki.astype(jnp.float32)
    ku = k * u
    n = jnp.floor(ku + 0.5)
    f = ku - n
    half = n * 0.5
    sgn = 1.0 - 4.0 * (half - jnp.floor(half))
    rf = _sinpi(f) * sgn

    s = s_ref[:, :]
    sp = jax.lax.broadcasted_iota(jnp.int32, (eb, _NSP), 1)
    onehot = (s == sp).astype(jnp.float32)
    psw = jnp.dot(onehot, wc_ref[:, :].T,
                  preferred_element_type=jnp.float32)
    gamma = psw * cutoff

    for a in range(_A):
        g = gamma[:, a][:, None]
        z = jnp.dot(rf, w1_ref[a], preferred_element_type=jnp.float32)
        h = _silu(z * g)
        h = _silu(jnp.dot(h, w2_ref[a], preferred_element_type=jnp.float32))
        h = _silu(jnp.dot(h, w3_ref[a], preferred_element_type=jnp.float32))
        o = jnp.dot(h, w4_ref[a], preferred_element_type=jnp.float32)
        out_ref[:, a, :] = o


def _block_diag_t(w):
    m = jnp.zeros((_A, _NTOT, _NTOT), jnp.float32)
    for l in range(_L):
        blk = jnp.transpose(w[l], (0, 2, 1))
        m = m.at[:, 32 * l:32 * (l + 1), 32 * l:32 * (l + 1)].set(blk)
    return m


@functools.partial(jax.jit, static_argnames=())
def kernel(r, species_neighbor, w_comb, mlp_w1, mlp_w2, mlp_w3, mlp_w4):
    e = r.shape[0]
    eb = 1280
    grid = pl.cdiv(e, eb)
    w1 = _block_diag_t(mlp_w1)
    w2 = _block_diag_t(mlp_w2)
    w3 = _block_diag_t(mlp_w3)
    w4 = _block_diag_t(mlp_w4)
    r2 = r.astype(jnp.float32).reshape(e, 1)
    s2 = species_neighbor.astype(jnp.int32).reshape(e, 1)
    full = lambda i: (0, 0, 0)
    return pl.pallas_call(
        _fwd,
        grid=(grid,),
        in_specs=[
            pl.BlockSpec((eb, 1), lambda i: (i, 0)),
            pl.BlockSpec((eb, 1), lambda i: (i, 0)),
            pl.BlockSpec((_A, _NSP), lambda i: (0, 0)),
            pl.BlockSpec((_A, _NTOT, _NTOT), full),
            pl.BlockSpec((_A, _NTOT, _NTOT), full),
            pl.BlockSpec((_A, _NTOT, _NTOT), full),
            pl.BlockSpec((_A, _NTOT, _NTOT), full),
        ],
        out_specs=pl.BlockSpec((eb, _A, _NTOT), lambda i: (i, 0, 0)),
        out_shape=jax.ShapeDtypeStruct((e, _A, _NTOT), jnp.float32),
    )(r2, s2, w_comb, w1, w2, w3, w4)
